# baseline (speedup 1.0000x reference)
import functools

import jax
import jax.numpy as jnp
from jax import lax
from jax.experimental import pallas as pl
from jax.experimental.pallas import tpu as pltpu
from jax.experimental.pallas import tpu_sc as plsc

B = 2048
NC = 16
D = 1024
H = 4096
TM = 128
NWORK = 32
CHUNK = B // NWORK


def _routing_body(cid_ref, dest_ref, offs_ref, oh_ref, run_ref):
    r = pl.program_id(0)

    @pl.when(r == 0)
    def _init():
        cid = cid_ref[:, :]
        cls = lax.broadcasted_iota(jnp.int32, (B, 128), 1)
        oh = (cid == cls).astype(jnp.float32)
        oh_ref[:, :] = oh
        counts = jnp.sum(oh, axis=0, keepdims=True)
        ii = lax.broadcasted_iota(jnp.int32, (128, 128), 0)
        jj = lax.broadcasted_iota(jnp.int32, (128, 128), 1)
        strict_lower = (ii < jj).astype(jnp.float32)
        offs = jnp.dot(counts, strict_lower,
                       preferred_element_type=jnp.float32)
        offs_ref[:, :] = offs.astype(jnp.int32)
        run_ref[:, :] = jnp.zeros((1, 128), jnp.float32)

    row0 = r * TM
    oh_b = oh_ref[pl.ds(row0, TM), :]
    li = lax.broadcasted_iota(jnp.int32, (TM, TM), 0)
    lj = lax.broadcasted_iota(jnp.int32, (TM, TM), 1)
    ltri = (lj <= li).astype(jnp.float32)
    prefix = (jnp.dot(ltri, oh_b, preferred_element_type=jnp.float32)
              + run_ref[:, :])
    run_ref[:, :] = prefix[TM - 1:TM, :]
    offs_f = offs_ref[:, :].astype(jnp.float32)
    dest_b = jnp.sum(oh_b * (offs_f + prefix - 1.0),
                     axis=1, keepdims=True)
    dest_ref[pl.ds(row0, TM), :] = dest_b.astype(jnp.int32)


def _routing(cid):
    return pl.pallas_call(
        _routing_body,
        grid=(16,),
        in_specs=[pl.BlockSpec((B, 1), lambda r: (0, 0))],
        out_specs=[
            pl.BlockSpec((B, 1), lambda r: (0, 0)),
            pl.BlockSpec((1, 128), lambda r: (0, 0)),
        ],
        out_shape=[
            jax.ShapeDtypeStruct((B, 1), jnp.int32),
            jax.ShapeDtypeStruct((1, 128), jnp.int32),
        ],
        scratch_shapes=[
            pltpu.VMEM((B, 128), jnp.float32),
            pltpu.VMEM((1, 128), jnp.float32),
        ],
    )(cid)


def _make_sc_gather():
    mesh = plsc.VectorSubcoreMesh(core_axis_name="c", subcore_axis_name="s")

    @functools.partial(
        pl.kernel,
        mesh=mesh,
        out_type=jax.ShapeDtypeStruct((B, D), jnp.float32),
        scratch_types=[
            pltpu.VMEM((CHUNK,), jnp.int32),
            pltpu.VMEM((CHUNK, D), jnp.float32),
            pltpu.SemaphoreType.DMA,
            pltpu.SemaphoreType.DMA,
            pltpu.SemaphoreType.DMA,
        ],
    )
    def gather_k(table_hbm, idx_hbm, out_hbm, idx_v, rows_v, sem, sem2,
                 wsem):
        wid = lax.axis_index("s") * 2 + lax.axis_index("c")
        base = wid * CHUNK
        half = CHUNK // 2
        pltpu.sync_copy(idx_hbm.at[pl.ds(base, CHUNK)], idx_v)
        g0 = pltpu.async_copy(
            table_hbm.at[idx_v.at[pl.ds(0, half)]],
            rows_v.at[pl.ds(0, half)], sem)
        g1 = pltpu.async_copy(
            table_hbm.at[idx_v.at[pl.ds(half, half)]],
            rows_v.at[pl.ds(half, half)], sem2)
        g0.wait()
        w0 = pltpu.async_copy(
            rows_v.at[pl.ds(0, half)],
            out_hbm.at[pl.ds(base, half)], wsem)
        g1.wait()
        w1 = pltpu.async_copy(
            rows_v.at[pl.ds(half, half)],
            out_hbm.at[pl.ds(base + half, half)], wsem)
        w0.wait()
        w1.wait()

    return gather_k


def _make_sc_scatter_f32():
    mesh = plsc.VectorSubcoreMesh(core_axis_name="c", subcore_axis_name="s")

    @functools.partial(
        pl.kernel,
        mesh=mesh,
        out_type=jax.ShapeDtypeStruct((B, D), jnp.float32),
        scratch_types=[
            pltpu.VMEM((CHUNK // 2,), jnp.int32),
            pltpu.VMEM((CHUNK // 2,), jnp.int32),
            pltpu.VMEM((CHUNK, D), jnp.float32),
            pltpu.SemaphoreType.DMA,
            pltpu.SemaphoreType.DMA,
            pltpu.SemaphoreType.DMA,
            pltpu.SemaphoreType.DMA,
        ],
    )
    def scatter_k(table_hbm, idx_hbm, out_hbm, idx_v0, idx_v1, rows_v,
                  sem, sem2, wsem, wsem2):
        wid = lax.axis_index("s") * 2 + lax.axis_index("c")
        base = wid * CHUNK
        half = CHUNK // 2
        pltpu.sync_copy(idx_hbm.at[pl.ds(base, half)], idx_v0)
        pltpu.sync_copy(idx_hbm.at[pl.ds(base + half, half)], idx_v1)
        l0 = pltpu.async_copy(
            table_hbm.at[pl.ds(base, half)],
            rows_v.at[pl.ds(0, half)], sem)
        l1 = pltpu.async_copy(
            table_hbm.at[pl.ds(base + half, half)],
            rows_v.at[pl.ds(half, half)], sem2)
        l0.wait()
        s0 = pltpu.async_copy(
            rows_v.at[pl.ds(0, half)], out_hbm.at[idx_v0], wsem)
        l1.wait()
        s1 = pltpu.async_copy(
            rows_v.at[pl.ds(half, half)], out_hbm.at[idx_v1], wsem2)
        s0.wait()
        s1.wait()

    return scatter_k


HC = H // 2
KH = H // HC


def _aligned_base(start, t):
    base = jnp.minimum(start + t * TM, B - TM)
    return pl.multiple_of((base // 8) * 8, 8)


def _mlp_body(offs_ref, zs_ref, w1_ref, b1_ref, w2_ref, b2_ref, ys_ref):
    i = pl.program_id(0)
    k = pl.program_id(1)
    start = offs_ref[i]
    end = offs_ref[i + 1]
    cnt = end - start
    nt = lax.div(cnt + 7 + TM - 1, TM)
    w1 = w1_ref[0]
    b1 = b1_ref[0]
    w2 = w2_ref[0]
    b2 = b2_ref[0]

    def tile(t, carry):
        base = _aligned_base(start, t)
        prev_end = _aligned_base(start, t - 1) + TM

        @pl.when((t == 0) | (base + TM > prev_end))
        def _():
            zt = zs_ref[pl.ds(base, TM), :].astype(jnp.float32)
            h = jnp.maximum(
                jnp.dot(zt, w1, preferred_element_type=jnp.float32) + b1,
                0.0)
            part = jnp.dot(h, w2, preferred_element_type=jnp.float32)
            rid = base + lax.broadcasted_iota(jnp.int32, (TM, 1), 0)
            m = (rid >= start) & (rid < end) & ((t == 0) | (rid >= prev_end))
            cur = ys_ref[pl.ds(base, TM), :]
            y = jnp.where(k == 0, part + b2, cur + part)
            ys_ref[pl.ds(base, TM), :] = jnp.where(m, y, cur)

        return carry

    lax.fori_loop(0, nt, tile, 0)


def _grouped_mlp(offs, zs, W1, b1, W2, b2):
    return pl.pallas_call(
        _mlp_body,
        grid=(NC, KH),
        in_specs=[
            pl.BlockSpec(memory_space=pltpu.SMEM),
            pl.BlockSpec((B, D), lambda i, k: (0, 0)),
            pl.BlockSpec((1, D, HC), lambda i, k: (i, 0, k)),
            pl.BlockSpec((1, 1, HC), lambda i, k: (i, 0, k)),
            pl.BlockSpec((1, HC, D), lambda i, k: (i, k, 0)),
            pl.BlockSpec((1, 1, D), lambda i, k: (i, 0, 0)),
        ],
        out_specs=pl.BlockSpec((B, D), lambda i, k: (0, 0)),
        out_shape=jax.ShapeDtypeStruct((B, D), jnp.float32),
        compiler_params=pltpu.CompilerParams(
            vmem_limit_bytes=128 * 1024 * 1024,
        ),
    )(offs, zs, W1, b1, W2, b2)


def kernel(z3d, class_ids, W1, b1, W2, b2):
    cid = class_ids.astype(jnp.int32).reshape(B, 1)
    dest, offs = _routing(cid)
    dest_flat = dest.reshape(B)
    zs = _make_sc_scatter_f32()(z3d, dest_flat).astype(jnp.bfloat16)
    ys = _grouped_mlp(
        offs.reshape(128), zs,
        W1, b1.reshape(NC, 1, H), W2, b2.reshape(NC, 1, D))
    return _make_sc_gather()(ys, dest_flat)


# speedup vs baseline: 1.0016x; 1.0016x over previous
import functools

import jax
import jax.numpy as jnp
from jax import lax
from jax.experimental import pallas as pl
from jax.experimental.pallas import tpu as pltpu
from jax.experimental.pallas import tpu_sc as plsc

B = 2048
NC = 16
D = 1024
H = 4096
TM = 128
NWORK = 32
CHUNK = B // NWORK


def _routing_body(cid_ref, dest_ref, offs_ref, oh_ref, run_ref):
    r = pl.program_id(0)

    @pl.when(r == 0)
    def _init():
        cid = cid_ref[:, :]
        cls = lax.broadcasted_iota(jnp.int32, (B, 128), 1)
        oh = (cid == cls).astype(jnp.float32)
        oh_ref[:, :] = oh
        counts = jnp.sum(oh, axis=0, keepdims=True)
        ii = lax.broadcasted_iota(jnp.int32, (128, 128), 0)
        jj = lax.broadcasted_iota(jnp.int32, (128, 128), 1)
        strict_lower = (ii < jj).astype(jnp.float32)
        offs = jnp.dot(counts, strict_lower,
                       preferred_element_type=jnp.float32)
        offs_ref[:, :] = offs.astype(jnp.int32)
        run_ref[:, :] = jnp.zeros((1, 128), jnp.float32)

    row0 = r * TM
    oh_b = oh_ref[pl.ds(row0, TM), :]
    li = lax.broadcasted_iota(jnp.int32, (TM, TM), 0)
    lj = lax.broadcasted_iota(jnp.int32, (TM, TM), 1)
    ltri = (lj <= li).astype(jnp.float32)
    prefix = (jnp.dot(ltri, oh_b, preferred_element_type=jnp.float32)
              + run_ref[:, :])
    run_ref[:, :] = prefix[TM - 1:TM, :]
    offs_f = offs_ref[:, :].astype(jnp.float32)
    dest_b = jnp.sum(oh_b * (offs_f + prefix - 1.0),
                     axis=1, keepdims=True)
    dest_ref[pl.ds(row0, TM), :] = dest_b.astype(jnp.int32)


def _routing(cid):
    return pl.pallas_call(
        _routing_body,
        grid=(16,),
        in_specs=[pl.BlockSpec((B, 1), lambda r: (0, 0))],
        out_specs=[
            pl.BlockSpec((B, 1), lambda r: (0, 0)),
            pl.BlockSpec((1, 128), lambda r: (0, 0)),
        ],
        out_shape=[
            jax.ShapeDtypeStruct((B, 1), jnp.int32),
            jax.ShapeDtypeStruct((1, 128), jnp.int32),
        ],
        scratch_shapes=[
            pltpu.VMEM((B, 128), jnp.float32),
            pltpu.VMEM((1, 128), jnp.float32),
        ],
    )(cid)


def _make_sc_gather():
    mesh = plsc.VectorSubcoreMesh(core_axis_name="c", subcore_axis_name="s")

    @functools.partial(
        pl.kernel,
        mesh=mesh,
        out_type=jax.ShapeDtypeStruct((B, D), jnp.float32),
        scratch_types=[
            pltpu.VMEM((CHUNK,), jnp.int32),
            pltpu.VMEM((CHUNK, D), jnp.float32),
            pltpu.SemaphoreType.DMA,
            pltpu.SemaphoreType.DMA,
            pltpu.SemaphoreType.DMA,
        ],
    )
    def gather_k(table_hbm, idx_hbm, out_hbm, idx_v, rows_v, sem, sem2,
                 wsem):
        wid = lax.axis_index("s") * 2 + lax.axis_index("c")
        base = wid * CHUNK
        half = CHUNK // 2
        pltpu.sync_copy(idx_hbm.at[pl.ds(base, CHUNK)], idx_v)
        g0 = pltpu.async_copy(
            table_hbm.at[idx_v.at[pl.ds(0, half)]],
            rows_v.at[pl.ds(0, half)], sem)
        g1 = pltpu.async_copy(
            table_hbm.at[idx_v.at[pl.ds(half, half)]],
            rows_v.at[pl.ds(half, half)], sem2)
        g0.wait()
        w0 = pltpu.async_copy(
            rows_v.at[pl.ds(0, half)],
            out_hbm.at[pl.ds(base, half)], wsem)
        g1.wait()
        w1 = pltpu.async_copy(
            rows_v.at[pl.ds(half, half)],
            out_hbm.at[pl.ds(base + half, half)], wsem)
        w0.wait()
        w1.wait()

    return gather_k


def _make_sc_scatter_f32():
    mesh = plsc.VectorSubcoreMesh(core_axis_name="c", subcore_axis_name="s")

    @functools.partial(
        pl.kernel,
        mesh=mesh,
        out_type=jax.ShapeDtypeStruct((B, D), jnp.float32),
        scratch_types=[
            pltpu.VMEM((CHUNK // 2,), jnp.int32),
            pltpu.VMEM((CHUNK // 2,), jnp.int32),
            pltpu.VMEM((CHUNK, D), jnp.float32),
            pltpu.SemaphoreType.DMA,
            pltpu.SemaphoreType.DMA,
            pltpu.SemaphoreType.DMA,
            pltpu.SemaphoreType.DMA,
        ],
    )
    def scatter_k(table_hbm, idx_hbm, out_hbm, idx_v0, idx_v1, rows_v,
                  sem, sem2, wsem, wsem2):
        wid = lax.axis_index("s") * 2 + lax.axis_index("c")
        base = wid * CHUNK
        half = CHUNK // 2
        pltpu.sync_copy(idx_hbm.at[pl.ds(base, half)], idx_v0)
        pltpu.sync_copy(idx_hbm.at[pl.ds(base + half, half)], idx_v1)
        l0 = pltpu.async_copy(
            table_hbm.at[pl.ds(base, half)],
            rows_v.at[pl.ds(0, half)], sem)
        l1 = pltpu.async_copy(
            table_hbm.at[pl.ds(base + half, half)],
            rows_v.at[pl.ds(half, half)], sem2)
        l0.wait()
        s0 = pltpu.async_copy(
            rows_v.at[pl.ds(0, half)], out_hbm.at[idx_v0], wsem)
        l1.wait()
        s1 = pltpu.async_copy(
            rows_v.at[pl.ds(half, half)], out_hbm.at[idx_v1], wsem2)
        s0.wait()
        s1.wait()

    return scatter_k


HC = H // 2
KH = H // HC


def _aligned_base(start, t):
    base = jnp.minimum(start + t * TM, B - TM)
    return pl.multiple_of((base // 8) * 8, 8)


def _mlp_body(offs_ref, zs_ref, w1_ref, b1_ref, w2_ref, b2_ref, ys_ref):
    i = pl.program_id(0)
    k = pl.program_id(1)
    start = offs_ref[i]
    end = offs_ref[i + 1]
    cnt = end - start
    nt = lax.div(cnt + 7 + TM - 1, TM)
    w1 = w1_ref[0].astype(jnp.bfloat16)
    b1 = b1_ref[0]
    w2 = w2_ref[0].astype(jnp.bfloat16)
    b2 = b2_ref[0]

    def tile(t, carry):
        base = _aligned_base(start, t)
        prev_end = _aligned_base(start, t - 1) + TM

        @pl.when((t == 0) | (base + TM > prev_end))
        def _():
            zt = zs_ref[pl.ds(base, TM), :]
            h = jnp.maximum(
                jnp.dot(zt, w1, preferred_element_type=jnp.float32) + b1,
                0.0)
            part = jnp.dot(h.astype(jnp.bfloat16), w2,
                           preferred_element_type=jnp.float32)
            rid = base + lax.broadcasted_iota(jnp.int32, (TM, 1), 0)
            m = (rid >= start) & (rid < end) & ((t == 0) | (rid >= prev_end))
            cur = ys_ref[pl.ds(base, TM), :]
            y = jnp.where(k == 0, part + b2, cur + part)
            ys_ref[pl.ds(base, TM), :] = jnp.where(m, y, cur)

        return carry

    lax.fori_loop(0, nt, tile, 0)


def _grouped_mlp(offs, zs, W1, b1, W2, b2):
    return pl.pallas_call(
        _mlp_body,
        grid=(NC, KH),
        in_specs=[
            pl.BlockSpec(memory_space=pltpu.SMEM),
            pl.BlockSpec((B, D), lambda i, k: (0, 0)),
            pl.BlockSpec((1, D, HC), lambda i, k: (i, 0, k)),
            pl.BlockSpec((1, 1, HC), lambda i, k: (i, 0, k)),
            pl.BlockSpec((1, HC, D), lambda i, k: (i, k, 0)),
            pl.BlockSpec((1, 1, D), lambda i, k: (i, 0, 0)),
        ],
        out_specs=pl.BlockSpec((B, D), lambda i, k: (0, 0)),
        out_shape=jax.ShapeDtypeStruct((B, D), jnp.float32),
        compiler_params=pltpu.CompilerParams(
            vmem_limit_bytes=128 * 1024 * 1024,
        ),
    )(offs, zs, W1, b1, W2, b2)


def kernel(z3d, class_ids, W1, b1, W2, b2):
    cid = class_ids.astype(jnp.int32).reshape(B, 1)
    dest, offs = _routing(cid)
    dest_flat = dest.reshape(B)
    zs = _make_sc_scatter_f32()(z3d, dest_flat).astype(jnp.bfloat16)
    ys = _grouped_mlp(
        offs.reshape(128), zs,
        W1, b1.reshape(NC, 1, H), W2, b2.reshape(NC, 1, D))
    return _make_sc_gather()(ys, dest_flat)
